# baseline (speedup 1.0000x reference)
import functools

import jax
import jax.numpy as jnp
from jax import lax
from jax.experimental import pallas as pl
from jax.experimental.pallas import tpu as pltpu
from jax.experimental.pallas import tpu_sc as plsc

B, N, L, D = 8, 2, 2048, 1024
NC, NS = 2, 16
BLKS = 4
TR = 32
MAXJ = 18
CH = 128
DC = D // 16

_mesh = plsc.VectorSubcoreMesh(
    core_axis_name="c", subcore_axis_name="s", num_cores=NC, num_subcores=NS
)


@functools.partial(
    pl.kernel,
    out_type=jax.ShapeDtypeStruct((B, D), jnp.float32),
    mesh=_mesh,
    scratch_types=[
        pltpu.VMEM((TR, D), jnp.float32),
        pltpu.VMEM((TR, D), jnp.float32),
        pltpu.VMEM((TR, D), jnp.float32),
        pltpu.VMEM((3 * TR,), jnp.float32),
        pltpu.VMEM((BLKS * D,), jnp.float32),
        pltpu.VMEM((16,), jnp.int32),
        pltpu.VMEM((2 * L,), jnp.int32),
        pltpu.VMEM((L,), jnp.int32),
        pltpu.VMEM((L,), jnp.int32),
        pltpu.VMEM((L + 16,), jnp.int32),
        pltpu.VMEM((L,), jnp.float32),
        pltpu.VMEM((NS * D // 2,), jnp.float32),
        pltpu.VMEM((D,), jnp.float32),
        pltpu.VMEM_SHARED((BLKS * L,), jnp.float32),
        pltpu.VMEM_SHARED((NS * BLKS * D,), jnp.float32),
        pltpu.SemaphoreType.DMA,
        pltpu.SemaphoreType.DMA,
        pltpu.SemaphoreType.DMA,
        pltpu.SemaphoreType.DMA,
        pltpu.SemaphoreType.DMA,
        pltpu.SemaphoreType.DMA,
        pltpu.SemaphoreType.DMA,
    ],
    compiler_params=pltpu.CompilerParams(needs_layout_passes=False),
)
def _vq_pool_kernel(feat, lens, vq, out, buf0, buf1, buf2, wbuf, acc4, lenv,
                    vqv, keyv, segv, startsv, wfull, psum8, outv, sh_w, sh_p,
                    sem0, sem1, sem2, wsem0, wsem1, wsem2, csem):
    c = lax.axis_index("c")
    s = lax.axis_index("s")
    bloc = s // BLKS
    b0 = c * BLKS

    NBUF = 3
    bufs = (buf0, buf1, buf2)
    sems = (sem0, sem1, sem2)
    wsems = (wsem0, wsem1, wsem2)

    pltpu.sync_copy(lens, lenv.at[pl.ds(0, B)])

    iota = lax.iota(jnp.int32, 16)
    zeros16 = jnp.full((16,), 0, jnp.int32)
    ones16 = jnp.full((16,), 1, jnp.int32)
    zf = jnp.full((16,), 0.0, jnp.float32)

    lens4 = plsc.load_gather(lenv, [b0 + (iota & 3)])
    mask4 = iota < 4
    ntv = jnp.where(mask4, (lens4 + (TR - 1)) >> 5, zeros16)
    pv = plsc.cumsum(ntv)
    total = jnp.sum(ntv)

    qw = jnp.maximum((total >> 4) - 4, 0)
    rem = total - BLKS * qw
    qn = ((rem + 11) * 2731) >> 15
    is_ww = (s % BLKS) == 0
    k_nw = s - 1 - (s // BLKS)
    start_g = jnp.where(is_ww, (s // BLKS) * qw, BLKS * qw + k_nw * qn)
    my_cnt = jnp.where(
        is_ww, qw,
        jnp.minimum(qn, jnp.maximum(total - start_g, 0)))

    def tile_of(g):
        cond = (pv <= g) & mask4
        ci = cond.astype(jnp.int32)
        bb = jnp.sum(ci)
        before = jnp.sum(jnp.where(cond, ntv, zeros16))
        return jnp.minimum(bb, BLKS - 1), jnp.minimum(g - before,
                                                      L // TR - 1)

    row_descs = [None, None, None]
    w_descs = [None, None, None]
    metas = [None, None, None]
    for j in range(NBUF):
        g = start_g + j
        bb, t = tile_of(g)
        metas[j] = (bb, t)
        d = pltpu.make_async_copy(
            feat.at[b0 + bb, 1, pl.ds(t * TR, TR), :], bufs[j], sems[j])

        @pl.when(j < my_cnt)
        def _(d=d):
            d.start()

        row_descs[j] = d

    lenvec = plsc.load_gather(lenv, [jnp.full((16,), b0 + bloc, jnp.int32)])

    @pl.when(s % BLKS == 0)
    def _weights():
        pltpu.sync_copy(vq.at[b0 + bloc], vqv)
        length_s = jnp.max(lenvec)
        nch = jnp.minimum((length_s + 46 + 15) >> 4, CH)

        def prefill(i, carry):
            plsc.store_scatter(startsv, [i * 16 + iota], lenvec)
            return carry

        lax.fori_loop(0, nch + 1, prefill, jnp.int32(0))

        def pass_a(i, seg_base):
            pos = i * 16 + iota
            k0 = plsc.load_gather(vqv, [2 * pos])
            k1 = plsc.load_gather(vqv, [2 * pos + ones16])
            key = k0 * 1024 + k1
            plsc.store_scatter(keyv, [pos], key)
            prev = plsc.load_gather(keyv, [jnp.maximum(pos - 1, zeros16)])
            chg = ((key != prev) | (pos == 0)) & (pos < lenvec)
            ci = chg.astype(jnp.int32)
            seg = seg_base + plsc.cumsum(ci) - 1
            plsc.store_scatter(segv, [pos], seg)
            plsc.store_scatter(startsv, [seg], pos, mask=chg)
            return seg_base + jnp.sum(ci)

        num_groups = lax.fori_loop(0, nch, pass_a, jnp.int32(0))
        gf = num_groups.astype(jnp.float32)

        def pass_b(i, carry):
            pos = i * 16 + iota
            seg = plsc.load_gather(segv, [pos])
            s0 = plsc.load_gather(startsv, [seg])
            s1 = plsc.load_gather(startsv, [seg + 1])
            cnt = (s1 - s0).astype(jnp.float32)
            w = jnp.where(pos < lenvec, 1.0 / (gf * cnt), 0.0)
            plsc.store_scatter(wfull, [pos], w)
            return carry

        lax.fori_loop(0, nch, pass_b, jnp.int32(0))
        pltpu.sync_copy(wfull, sh_w.at[pl.ds(pl.multiple_of(bloc * L, 8), L)])

    plsc.subcore_barrier()

    def zbody(i, carry):
        plsc.store_scatter(acc4, [i * 16 + iota], zf)
        return carry

    lax.fori_loop(0, BLKS * DC, zbody, jnp.int32(0))

    for j in range(NBUF):
        bb, t = metas[j]
        d = pltpu.make_async_copy(
            sh_w.at[pl.ds(pl.multiple_of(bb * L + t * TR, 8), TR)],
            wbuf.at[pl.ds(j * TR, TR)], wsems[j])

        @pl.when(j < my_cnt)
        def _(d=d):
            d.start()

        w_descs[j] = d

    KG = 16
    NG = DC // KG
    for j in range(MAXJ):
        bi = j % NBUF
        buf = bufs[bi]
        bb, t = metas[bi]
        abase = bb * D

        @pl.when(j < my_cnt)
        def _tile(bi=bi, buf=buf, abase=abase):
            row_descs[bi].wait()
            w_descs[bi].wait()
            for gr in range(NG):
                base = gr * KG * 16
                accs = tuple(
                    acc4[pl.ds(abase + base + k * 16, 16)]
                    for k in range(KG))

                def rbody(r, accs, bi=bi, buf=buf, base=base):
                    wv = plsc.load_gather(
                        wbuf, [jnp.full((16,), bi * TR, jnp.int32) + r])
                    return tuple(
                        a + wv * buf[r, pl.ds(base + k * 16, 16)]
                        for k, a in enumerate(accs))

                accs = lax.fori_loop(0, TR, rbody, accs)
                for k in range(KG):
                    acc4[pl.ds(abase + base + k * 16, 16)] = accs[k]

        if j + NBUF < MAXJ:
            g2 = start_g + (j + NBUF)
            bb2, t2 = tile_of(g2)
            metas[bi] = (bb2, t2)
            dr = pltpu.make_async_copy(
                feat.at[b0 + bb2, 1, pl.ds(t2 * TR, TR), :], buf, sems[bi])
            dw = pltpu.make_async_copy(
                sh_w.at[pl.ds(pl.multiple_of(bb2 * L + t2 * TR, 8), TR)],
                wbuf.at[pl.ds(bi * TR, TR)], wsems[bi])

            @pl.when(j + NBUF < my_cnt)
            def _start(dr=dr, dw=dw):
                dr.start()
                dw.start()

            row_descs[bi] = dr
            w_descs[bi] = dw

    pltpu.sync_copy(acc4, sh_p.at[pl.ds(pl.multiple_of(s * BLKS * D, 8),
                                        BLKS * D)])
    plsc.subcore_barrier()

    @pl.when(s < BLKS)
    def _combine():
        for rnd in range(2):
            cds = []
            for w in range(NS // 2):
                wg = rnd * (NS // 2) + w
                d = pltpu.make_async_copy(
                    sh_p.at[pl.ds(
                        pl.multiple_of((wg * BLKS) * D + s * D, 8), D)],
                    psum8.at[pl.ds(w * D, D)], csem)
                d.start()
                cds.append(d)
            for d in cds:
                d.wait()

            def cbody(dc, carry, rnd=rnd):
                tot = zf
                for w in range(NS // 2):
                    tot = tot + psum8[pl.ds(w * D + dc * 16, 16)]
                if rnd:
                    tot = tot + outv[pl.ds(dc * 16, 16)]
                outv[pl.ds(dc * 16, 16)] = tot
                return carry

            lax.fori_loop(0, DC, cbody, jnp.int32(0))
        pltpu.sync_copy(outv, out.at[b0 + s])


def kernel(input_feature, input_lengths, vq_indices):
    return _vq_pool_kernel(
        input_feature, input_lengths, vq_indices.reshape(B, 2 * L))


# speedup vs baseline: 1.0714x; 1.0714x over previous
import functools

import jax
import jax.numpy as jnp
from jax import lax
from jax.experimental import pallas as pl
from jax.experimental.pallas import tpu as pltpu
from jax.experimental.pallas import tpu_sc as plsc

B, N, L, D = 8, 2, 2048, 1024
NC, NS = 2, 16
BLKS = 4
TR = 32
MAXJ = BLKS * (L // TR) // NS
CH = 128
DC = D // 16

_mesh = plsc.VectorSubcoreMesh(
    core_axis_name="c", subcore_axis_name="s", num_cores=NC, num_subcores=NS
)


@functools.partial(
    pl.kernel,
    out_type=jax.ShapeDtypeStruct((B, D), jnp.float32),
    mesh=_mesh,
    scratch_types=[
        pltpu.VMEM((TR, D), jnp.float32),
        pltpu.VMEM((TR, D), jnp.float32),
        pltpu.VMEM((TR, D), jnp.float32),
        pltpu.VMEM((3 * TR,), jnp.float32),
        pltpu.VMEM((BLKS * D,), jnp.float32),
        pltpu.VMEM((16,), jnp.int32),
        pltpu.VMEM((2 * L,), jnp.int32),
        pltpu.VMEM((L,), jnp.int32),
        pltpu.VMEM((L,), jnp.int32),
        pltpu.VMEM((L + 16,), jnp.int32),
        pltpu.VMEM((L,), jnp.float32),
        pltpu.VMEM((NS * D // 2,), jnp.float32),
        pltpu.VMEM((D,), jnp.float32),
        pltpu.VMEM_SHARED((BLKS * L,), jnp.float32),
        pltpu.VMEM_SHARED((NS * BLKS * D,), jnp.float32),
        pltpu.SemaphoreType.DMA,
        pltpu.SemaphoreType.DMA,
        pltpu.SemaphoreType.DMA,
        pltpu.SemaphoreType.DMA,
        pltpu.SemaphoreType.DMA,
        pltpu.SemaphoreType.DMA,
        pltpu.SemaphoreType.DMA,
    ],
    compiler_params=pltpu.CompilerParams(needs_layout_passes=False),
)
def _vq_pool_kernel(feat, lens, vq, out, buf0, buf1, buf2, wbuf, acc4, lenv,
                    vqv, keyv, segv, startsv, wfull, psum8, outv, sh_w, sh_p,
                    sem0, sem1, sem2, wsem0, wsem1, wsem2, csem):
    c = lax.axis_index("c")
    s = lax.axis_index("s")
    bloc = s // BLKS
    b0 = c * BLKS

    NBUF = 3
    bufs = (buf0, buf1, buf2)
    sems = (sem0, sem1, sem2)
    wsems = (wsem0, wsem1, wsem2)

    pltpu.sync_copy(lens, lenv.at[pl.ds(0, B)])

    iota = lax.iota(jnp.int32, 16)
    zeros16 = jnp.full((16,), 0, jnp.int32)
    ones16 = jnp.full((16,), 1, jnp.int32)
    zf = jnp.full((16,), 0.0, jnp.float32)

    lens4 = plsc.load_gather(lenv, [b0 + (iota & 3)])
    mask4 = iota < 4
    ntv = jnp.where(mask4, (lens4 + (TR - 1)) >> 5, zeros16)
    pv = plsc.cumsum(ntv)
    total = jnp.sum(ntv)

    def tile_of(g):
        cond = (pv <= g) & mask4
        ci = cond.astype(jnp.int32)
        bb = jnp.sum(ci)
        before = jnp.sum(jnp.where(cond, ntv, zeros16))
        return jnp.minimum(bb, BLKS - 1), jnp.minimum(g - before,
                                                      L // TR - 1)

    row_descs = [None, None, None]
    w_descs = [None, None, None]
    metas = [None, None, None]
    for j in range(NBUF):
        g = s + j * NS
        bb, t = tile_of(g)
        metas[j] = (bb, t)
        d = pltpu.make_async_copy(
            feat.at[b0 + bb, 1, pl.ds(t * TR, TR), :], bufs[j], sems[j])

        @pl.when(g < total)
        def _(d=d):
            d.start()

        row_descs[j] = d

    lenvec = plsc.load_gather(lenv, [jnp.full((16,), b0 + bloc, jnp.int32)])

    @pl.when(s % BLKS == 0)
    def _weights():
        pltpu.sync_copy(vq.at[b0 + bloc], vqv)
        length_s = jnp.max(lenvec)
        nch = jnp.minimum((length_s + 46 + 15) >> 4, CH)

        def prefill(i, carry):
            plsc.store_scatter(startsv, [i * 16 + iota], lenvec)
            return carry

        lax.fori_loop(0, nch + 1, prefill, jnp.int32(0))

        def pass_a(i, seg_base):
            pos = i * 16 + iota
            k0 = plsc.load_gather(vqv, [2 * pos])
            k1 = plsc.load_gather(vqv, [2 * pos + ones16])
            key = k0 * 1024 + k1
            plsc.store_scatter(keyv, [pos], key)
            prev = plsc.load_gather(keyv, [jnp.maximum(pos - 1, zeros16)])
            chg = ((key != prev) | (pos == 0)) & (pos < lenvec)
            ci = chg.astype(jnp.int32)
            seg = seg_base + plsc.cumsum(ci) - 1
            plsc.store_scatter(segv, [pos], seg)
            plsc.store_scatter(startsv, [seg], pos, mask=chg)
            return seg_base + jnp.sum(ci)

        num_groups = lax.fori_loop(0, nch, pass_a, jnp.int32(0))
        gf = num_groups.astype(jnp.float32)

        def pass_b(i, carry):
            pos = i * 16 + iota
            seg = plsc.load_gather(segv, [pos])
            s0 = plsc.load_gather(startsv, [seg])
            s1 = plsc.load_gather(startsv, [seg + 1])
            cnt = (s1 - s0).astype(jnp.float32)
            w = jnp.where(pos < lenvec, 1.0 / (gf * cnt), 0.0)
            plsc.store_scatter(wfull, [pos], w)
            return carry

        lax.fori_loop(0, nch, pass_b, jnp.int32(0))
        pltpu.sync_copy(wfull, sh_w.at[pl.ds(pl.multiple_of(bloc * L, 8), L)])

    plsc.subcore_barrier()

    def zbody(i, carry):
        plsc.store_scatter(acc4, [i * 16 + iota], zf)
        return carry

    lax.fori_loop(0, BLKS * DC, zbody, jnp.int32(0))

    for j in range(NBUF):
        g = s + j * NS
        bb, t = metas[j]
        d = pltpu.make_async_copy(
            sh_w.at[pl.ds(pl.multiple_of(bb * L + t * TR, 8), TR)],
            wbuf.at[pl.ds(j * TR, TR)], wsems[j])

        @pl.when(g < total)
        def _(d=d):
            d.start()

        w_descs[j] = d

    KG = 16
    NG = DC // KG
    for j in range(MAXJ):
        bi = j % NBUF
        g = s + j * NS
        buf = bufs[bi]
        bb, t = metas[bi]
        abase = bb * D

        @pl.when(g < total)
        def _tile(bi=bi, buf=buf, abase=abase):
            row_descs[bi].wait()
            w_descs[bi].wait()
            for gr in range(NG):
                base = gr * KG * 16
                accs = tuple(
                    acc4[pl.ds(abase + base + k * 16, 16)]
                    for k in range(KG))

                def rbody(r, accs, bi=bi, buf=buf, base=base):
                    wv = plsc.load_gather(
                        wbuf, [jnp.full((16,), bi * TR, jnp.int32) + r])
                    return tuple(
                        a + wv * buf[r, pl.ds(base + k * 16, 16)]
                        for k, a in enumerate(accs))

                accs = lax.fori_loop(0, TR, rbody, accs)
                for k in range(KG):
                    acc4[pl.ds(abase + base + k * 16, 16)] = accs[k]

        if j + NBUF < MAXJ:
            g2 = s + (j + NBUF) * NS
            bb2, t2 = tile_of(g2)
            metas[bi] = (bb2, t2)
            dr = pltpu.make_async_copy(
                feat.at[b0 + bb2, 1, pl.ds(t2 * TR, TR), :], buf, sems[bi])
            dw = pltpu.make_async_copy(
                sh_w.at[pl.ds(pl.multiple_of(bb2 * L + t2 * TR, 8), TR)],
                wbuf.at[pl.ds(bi * TR, TR)], wsems[bi])

            @pl.when(g2 < total)
            def _start(dr=dr, dw=dw):
                dr.start()
                dw.start()

            row_descs[bi] = dr
            w_descs[bi] = dw

    pltpu.sync_copy(acc4, sh_p.at[pl.ds(pl.multiple_of(s * BLKS * D, 8),
                                        BLKS * D)])
    plsc.subcore_barrier()

    @pl.when(s < BLKS)
    def _combine():
        for rnd in range(2):
            cds = []
            for w in range(NS // 2):
                wg = rnd * (NS // 2) + w
                d = pltpu.make_async_copy(
                    sh_p.at[pl.ds(
                        pl.multiple_of((wg * BLKS) * D + s * D, 8), D)],
                    psum8.at[pl.ds(w * D, D)], csem)
                d.start()
                cds.append(d)
            for d in cds:
                d.wait()

            def cbody(dc, carry, rnd=rnd):
                tot = zf
                for w in range(NS // 2):
                    tot = tot + psum8[pl.ds(w * D + dc * 16, 16)]
                if rnd:
                    tot = tot + outv[pl.ds(dc * 16, 16)]
                outv[pl.ds(dc * 16, 16)] = tot
                return carry

            lax.fori_loop(0, DC, cbody, jnp.int32(0))
        pltpu.sync_copy(outv, out.at[b0 + s])


def kernel(input_feature, input_lengths, vq_indices):
    return _vq_pool_kernel(
        input_feature, input_lengths, vq_indices.reshape(B, 2 * L))
